# Initial kernel scaffold; baseline (speedup 1.0000x reference)
import jax
import jax.numpy as jnp
from jax.experimental import pallas as pl


def kernel(X, targets, emb):
    raise NotImplementedError("write your pallas kernel here")


# speedup vs baseline: 2.3891x; 2.3891x over previous
import functools

import jax
import jax.numpy as jnp
from jax.experimental import pallas as pl
from jax.experimental.pallas import tpu as pltpu

VOCAB_SIZE = 8192
NUM_ROWS = 8192
ROWS_PER_BLOCK = 128
NUM_BLOCKS = NUM_ROWS // ROWS_PER_BLOCK


def _fused_kernel(x_smem, tgt_ref, emb_hbm, out_hbm, loss_ref,
                  buf, in_sems, out_sems):
    i = pl.program_id(0)
    slot = jax.lax.rem(i, 2)
    next_slot = jax.lax.rem(i + 1, 2)

    def issue_in(block, dst_slot):
        def body(r, _):
            idx = x_smem[block * ROWS_PER_BLOCK + r]
            pltpu.make_async_copy(
                emb_hbm.at[idx],
                buf.at[dst_slot, r],
                in_sems.at[dst_slot],
            ).start()
            return 0
        jax.lax.fori_loop(0, ROWS_PER_BLOCK, body, 0)

    @pl.when(i == 0)
    def _():
        loss_ref[0, 0] = 0.0
        issue_in(0, 0)

    @pl.when(i >= 1)
    def _():
        pltpu.make_async_copy(
            buf.at[next_slot],
            out_hbm.at[pl.ds(0, ROWS_PER_BLOCK)],
            out_sems.at[next_slot],
        ).wait()

    @pl.when(i + 1 < NUM_BLOCKS)
    def _():
        issue_in(i + 1, next_slot)

    def wait_body(r, _):
        pltpu.make_async_copy(
            emb_hbm.at[0],
            buf.at[slot, r],
            in_sems.at[slot],
        ).wait()
        return 0
    jax.lax.fori_loop(0, ROWS_PER_BLOCK, wait_body, 0)

    rows = buf[slot]
    m = jnp.max(rows, axis=1, keepdims=True)
    s = jnp.sum(jnp.exp(rows - m), axis=1, keepdims=True)
    lse = jnp.log(s) + m
    tgt = tgt_ref[0, 0, :]
    col = jax.lax.broadcasted_iota(jnp.int32, rows.shape, 1)
    tl = jnp.sum(jnp.where(col == tgt[:, None], rows, 0.0), axis=1,
                 keepdims=True)
    loss_ref[0, 0] += jnp.sum(lse - tl) * (1.0 / NUM_ROWS)

    pltpu.make_async_copy(
        buf.at[slot],
        out_hbm.at[pl.ds(i * ROWS_PER_BLOCK, ROWS_PER_BLOCK)],
        out_sems.at[slot],
    ).start()

    @pl.when(i == NUM_BLOCKS - 1)
    def _():
        pltpu.make_async_copy(
            buf.at[slot],
            out_hbm.at[pl.ds(0, ROWS_PER_BLOCK)],
            out_sems.at[slot],
        ).wait()


@jax.jit
def _run(x_flat, tgt3, emb):
    grid_spec = pltpu.PrefetchScalarGridSpec(
        num_scalar_prefetch=1,
        grid=(NUM_BLOCKS,),
        in_specs=[
            pl.BlockSpec((1, 1, ROWS_PER_BLOCK), lambda i, X: (i, 0, 0)),
            pl.BlockSpec(memory_space=pl.ANY),
        ],
        out_specs=[
            pl.BlockSpec(memory_space=pl.ANY),
            pl.BlockSpec((1, 1), lambda i, X: (0, 0),
                         memory_space=pltpu.MemorySpace.SMEM),
        ],
        scratch_shapes=[
            pltpu.VMEM((2, ROWS_PER_BLOCK, VOCAB_SIZE), jnp.float32),
            pltpu.SemaphoreType.DMA((2,)),
            pltpu.SemaphoreType.DMA((2,)),
        ],
    )
    logits2, loss = pl.pallas_call(
        _fused_kernel,
        grid_spec=grid_spec,
        out_shape=[
            jax.ShapeDtypeStruct((NUM_ROWS, VOCAB_SIZE), jnp.float32),
            jax.ShapeDtypeStruct((1, 1), jnp.float32),
        ],
    )(x_flat, tgt3, emb)
    return logits2, loss[0, 0]


def kernel(X, targets, emb):
    x_flat = X.reshape(-1).astype(jnp.int32)
    tgt3 = targets.reshape(NUM_BLOCKS, 1, ROWS_PER_BLOCK).astype(jnp.int32)
    return _run(x_flat, tgt3, emb)


# speedup vs baseline: 3.1432x; 1.3156x over previous
import functools

import jax
import jax.numpy as jnp
from jax.experimental import pallas as pl
from jax.experimental.pallas import tpu as pltpu

VOCAB_SIZE = 8192
NUM_ROWS = 8192
ROWS_PER_BLOCK = 128
NUM_BLOCKS = NUM_ROWS // ROWS_PER_BLOCK


def _fused_kernel(x_smem, tgt_ref, emb_hbm, out_hbm, loss_ref,
                  buf, in_sems, out_sems):
    i = pl.program_id(0)
    slot = jax.lax.rem(i, 2)
    next_slot = jax.lax.rem(i + 1, 2)

    def issue_in(block, dst_slot):
        base = block * ROWS_PER_BLOCK
        unroll = 8
        def body(r8, _):
            r = r8 * unroll
            for u in range(unroll):
                idx = x_smem[base + r + u]
                pltpu.make_async_copy(
                    emb_hbm.at[idx],
                    buf.at[dst_slot, r + u],
                    in_sems.at[dst_slot],
                ).start()
            return 0
        jax.lax.fori_loop(0, ROWS_PER_BLOCK // unroll, body, 0)

    @pl.when(i == 0)
    def _():
        loss_ref[0, 0] = 0.0
        issue_in(0, 0)

    @pl.when(i >= 1)
    def _():
        pltpu.make_async_copy(
            buf.at[next_slot],
            out_hbm.at[pl.ds(0, ROWS_PER_BLOCK)],
            out_sems.at[next_slot],
        ).wait()

    @pl.when(i + 1 < NUM_BLOCKS)
    def _():
        issue_in(i + 1, next_slot)

    pltpu.make_async_copy(
        emb_hbm.at[pl.ds(0, ROWS_PER_BLOCK)],
        buf.at[slot],
        in_sems.at[slot],
    ).wait()

    rows = buf[slot]
    m = jnp.max(rows, axis=1, keepdims=True)
    s = jnp.sum(jnp.exp(rows - m), axis=1, keepdims=True)
    lse = jnp.log(s) + m
    tgt = tgt_ref[0, 0, :]
    col = jax.lax.broadcasted_iota(jnp.int32, rows.shape, 1)
    tl = jnp.sum(jnp.where(col == tgt[:, None], rows, 0.0), axis=1,
                 keepdims=True)
    loss_ref[0, 0] += jnp.sum(lse - tl) * (1.0 / NUM_ROWS)

    pltpu.make_async_copy(
        buf.at[slot],
        out_hbm.at[pl.ds(i * ROWS_PER_BLOCK, ROWS_PER_BLOCK)],
        out_sems.at[slot],
    ).start()

    @pl.when(i == NUM_BLOCKS - 1)
    def _():
        pltpu.make_async_copy(
            buf.at[slot],
            out_hbm.at[pl.ds(0, ROWS_PER_BLOCK)],
            out_sems.at[slot],
        ).wait()


@jax.jit
def _run(x_flat, tgt3, emb):
    grid_spec = pltpu.PrefetchScalarGridSpec(
        num_scalar_prefetch=1,
        grid=(NUM_BLOCKS,),
        in_specs=[
            pl.BlockSpec((1, 1, ROWS_PER_BLOCK), lambda i, X: (i, 0, 0)),
            pl.BlockSpec(memory_space=pl.ANY),
        ],
        out_specs=[
            pl.BlockSpec(memory_space=pl.ANY),
            pl.BlockSpec((1, 1), lambda i, X: (0, 0),
                         memory_space=pltpu.MemorySpace.SMEM),
        ],
        scratch_shapes=[
            pltpu.VMEM((2, ROWS_PER_BLOCK, VOCAB_SIZE), jnp.float32),
            pltpu.SemaphoreType.DMA((2,)),
            pltpu.SemaphoreType.DMA((2,)),
        ],
    )
    logits2, loss = pl.pallas_call(
        _fused_kernel,
        grid_spec=grid_spec,
        out_shape=[
            jax.ShapeDtypeStruct((NUM_ROWS, VOCAB_SIZE), jnp.float32),
            jax.ShapeDtypeStruct((1, 1), jnp.float32),
        ],
    )(x_flat, tgt3, emb)
    return logits2, loss[0, 0]


def kernel(X, targets, emb):
    x_flat = X.reshape(-1).astype(jnp.int32)
    tgt3 = targets.reshape(NUM_BLOCKS, 1, ROWS_PER_BLOCK).astype(jnp.int32)
    return _run(x_flat, tgt3, emb)


# speedup vs baseline: 3.8153x; 1.2139x over previous
import functools

import jax
import jax.numpy as jnp
from jax.experimental import pallas as pl
from jax.experimental.pallas import tpu as pltpu

VOCAB_SIZE = 8192
NUM_ROWS = 8192
ROWS_PER_BLOCK = 128
NUM_BLOCKS = NUM_ROWS // ROWS_PER_BLOCK


NUM_SLOTS = 4


def _fused_kernel(x_smem, tgt_ref, emb_hbm, out_hbm, loss_ref,
                  buf, in_sems, out_sems):
    i = pl.program_id(0)
    slot = jax.lax.rem(i, NUM_SLOTS)
    ahead_slot = jax.lax.rem(i + 2, NUM_SLOTS)

    def issue_in(block, dst_slot):
        base = block * ROWS_PER_BLOCK
        unroll = 8
        def body(r8, _):
            r = r8 * unroll
            for u in range(unroll):
                idx = x_smem[base + r + u]
                pltpu.make_async_copy(
                    emb_hbm.at[idx],
                    buf.at[dst_slot, r + u],
                    in_sems.at[dst_slot],
                ).start()
            return 0
        jax.lax.fori_loop(0, ROWS_PER_BLOCK // unroll, body, 0)

    @pl.when(i == 0)
    def _():
        loss_ref[0, 0] = 0.0
        issue_in(0, 0)
        issue_in(1, 1)

    @pl.when(i + 2 < NUM_BLOCKS)
    def _():
        @pl.when(i >= 2)
        def _():
            pltpu.make_async_copy(
                buf.at[ahead_slot],
                out_hbm.at[pl.ds(0, ROWS_PER_BLOCK)],
                out_sems.at[ahead_slot],
            ).wait()
        issue_in(i + 2, ahead_slot)

    pltpu.make_async_copy(
        emb_hbm.at[pl.ds(0, ROWS_PER_BLOCK)],
        buf.at[slot],
        in_sems.at[slot],
    ).wait()

    rows = buf[slot]
    m = jnp.max(rows, axis=1, keepdims=True)
    s = jnp.sum(jnp.exp(rows - m), axis=1, keepdims=True)
    lse = jnp.log(s) + m
    tgt = tgt_ref[0, 0, :]
    col = jax.lax.broadcasted_iota(jnp.int32, rows.shape, 1)
    tl = jnp.sum(jnp.where(col == tgt[:, None], rows, 0.0), axis=1,
                 keepdims=True)
    loss_ref[0, 0] += jnp.sum(lse - tl) * (1.0 / NUM_ROWS)

    pltpu.make_async_copy(
        buf.at[slot],
        out_hbm.at[pl.ds(i * ROWS_PER_BLOCK, ROWS_PER_BLOCK)],
        out_sems.at[slot],
    ).start()

    @pl.when(i == NUM_BLOCKS - 1)
    def _():
        for s in range(NUM_SLOTS):
            pltpu.make_async_copy(
                buf.at[s],
                out_hbm.at[pl.ds(0, ROWS_PER_BLOCK)],
                out_sems.at[s],
            ).wait()


@jax.jit
def _run(x_flat, tgt3, emb):
    grid_spec = pltpu.PrefetchScalarGridSpec(
        num_scalar_prefetch=1,
        grid=(NUM_BLOCKS,),
        in_specs=[
            pl.BlockSpec((1, 1, ROWS_PER_BLOCK), lambda i, X: (i, 0, 0)),
            pl.BlockSpec(memory_space=pl.ANY),
        ],
        out_specs=[
            pl.BlockSpec(memory_space=pl.ANY),
            pl.BlockSpec((1, 1), lambda i, X: (0, 0),
                         memory_space=pltpu.MemorySpace.SMEM),
        ],
        scratch_shapes=[
            pltpu.VMEM((NUM_SLOTS, ROWS_PER_BLOCK, VOCAB_SIZE), jnp.float32),
            pltpu.SemaphoreType.DMA((NUM_SLOTS,)),
            pltpu.SemaphoreType.DMA((NUM_SLOTS,)),
        ],
    )
    logits2, loss = pl.pallas_call(
        _fused_kernel,
        grid_spec=grid_spec,
        out_shape=[
            jax.ShapeDtypeStruct((NUM_ROWS, VOCAB_SIZE), jnp.float32),
            jax.ShapeDtypeStruct((1, 1), jnp.float32),
        ],
    )(x_flat, tgt3, emb)
    return logits2, loss[0, 0]


def kernel(X, targets, emb):
    x_flat = X.reshape(-1).astype(jnp.int32)
    tgt3 = targets.reshape(NUM_BLOCKS, 1, ROWS_PER_BLOCK).astype(jnp.int32)
    return _run(x_flat, tgt3, emb)
